# baseline (speedup 1.0000x reference)
import functools

import jax
import jax.numpy as jnp
from jax import lax
from jax.experimental import pallas as pl
from jax.experimental.pallas import tpu as pltpu
from jax.experimental.pallas import tpu_sc as plsc

D = 128
_R = 1000


def _ln(h, g, b):
    m = jnp.mean(h, axis=-1, keepdims=True)
    v = jnp.mean((h - m) ** 2, axis=-1, keepdims=True)
    return (h - m) * lax.rsqrt(v + 1e-5) * g + b


def _mlp_body(x_ref, a_ref, W1_ref, W2_ref, b1_ref, ng_ref, nb_ref, W3_ref,
              b3_ref, og_ref, ob_ref, o_ref):
    h = (jnp.dot(x_ref[...], W1_ref[...], preferred_element_type=jnp.float32)
         + jnp.dot(a_ref[...], W2_ref[...], preferred_element_type=jnp.float32)
         + b1_ref[...])
    h = _ln(h, ng_ref[...], nb_ref[...])
    h = jnp.maximum(h, 0.0)
    h = jnp.dot(h, W3_ref[...], preferred_element_type=jnp.float32) + b3_ref[...]
    h = _ln(h, og_ref[...], ob_ref[...])
    o_ref[...] = jnp.maximum(h, 0.0)


def _mlp_top_body(x_ref, a_ref, W1_ref, W2_ref, b1_ref, ng_ref, nb_ref, W3_ref,
                  b3_ref, og_ref, ob_ref, eW2_ref, o_ref, o2_ref):
    _mlp_body(x_ref, a_ref, W1_ref, W2_ref, b1_ref, ng_ref, nb_ref, W3_ref,
              b3_ref, og_ref, ob_ref, o_ref)
    o2_ref[...] = jnp.dot(o_ref[...], eW2_ref[...],
                          preferred_element_type=jnp.float32)


def _e1_body(e_ref, W1_ref, b1_ref, o_ref):
    o_ref[...] = (jnp.dot(e_ref[...], W1_ref[...],
                          preferred_element_type=jnp.float32) + b1_ref[...])


def _edge_body(e1_ref, g_ref, ng_ref, nb_ref, W3_ref, b3_ref,
               bg_ref, bb_ref, o_ref):
    h = e1_ref[...] + g_ref[...]
    h = _ln(h, ng_ref[...], nb_ref[...])
    h = jnp.maximum(h, 0.0)
    h = jnp.dot(h, W3_ref[...], preferred_element_type=jnp.float32) + b3_ref[...]
    h = h * bg_ref[...] + bb_ref[...]
    o_ref[...] = jnp.maximum(h, 0.0)


_row_spec = pl.BlockSpec((_R, D), lambda i: (i, 0))
_w_spec = pl.BlockSpec((D, D), lambda i: (0, 0))
_v_spec = pl.BlockSpec((1, D), lambda i: (0, 0))


def _tc_mlp(x, agg, W1, W2, b1, ng, nb, W3, b3, og, ob):
    M = x.shape[0]
    return pl.pallas_call(
        _mlp_body,
        grid=(M // _R,),
        in_specs=[_row_spec, _row_spec, _w_spec, _w_spec, _v_spec, _v_spec,
                  _v_spec, _w_spec, _v_spec, _v_spec, _v_spec],
        out_specs=_row_spec,
        out_shape=jax.ShapeDtypeStruct((M, D), jnp.float32),
    )(x, agg, W1, W2, b1.reshape(1, D), ng.reshape(1, D), nb.reshape(1, D),
      W3, b3.reshape(1, D), og.reshape(1, D), ob.reshape(1, D))


def _tc_mlp_top(x, agg, W1, W2, b1, ng, nb, W3, b3, og, ob, eW2):
    M = x.shape[0]
    return pl.pallas_call(
        _mlp_top_body,
        grid=(M // _R,),
        in_specs=[_row_spec, _row_spec, _w_spec, _w_spec, _v_spec, _v_spec,
                  _v_spec, _w_spec, _v_spec, _v_spec, _v_spec, _w_spec],
        out_specs=[_row_spec, _row_spec],
        out_shape=[jax.ShapeDtypeStruct((M, D), jnp.float32),
                   jax.ShapeDtypeStruct((M, D), jnp.float32)],
    )(x, agg, W1, W2, b1.reshape(1, D), ng.reshape(1, D), nb.reshape(1, D),
      W3, b3.reshape(1, D), og.reshape(1, D), ob.reshape(1, D), eW2)


def _tc_e1(e_feat, W1, b1):
    M = e_feat.shape[0]
    return pl.pallas_call(
        _e1_body,
        grid=(M // _R,),
        in_specs=[_row_spec, _w_spec, _v_spec],
        out_specs=_row_spec,
        out_shape=jax.ShapeDtypeStruct((M, D), jnp.float32),
    )(e_feat, W1, b1.reshape(1, D))


def _tc_edge(e1, n2e2, ng, nb, W3, b3, bg, bb):
    M = e1.shape[0]
    return pl.pallas_call(
        _edge_body,
        grid=(M // _R,),
        in_specs=[_row_spec, _row_spec, _v_spec, _v_spec,
                  _w_spec, _v_spec, _v_spec, _v_spec],
        out_specs=_row_spec,
        out_shape=jax.ShapeDtypeStruct((M, D), jnp.float32),
    )(e1, n2e2, ng.reshape(1, D), nb.reshape(1, D),
      W3, b3.reshape(1, D), bg.reshape(1, D), bb.reshape(1, D))


_MESH = dict(
    mesh=plsc.VectorSubcoreMesh(core_axis_name="c", subcore_axis_name="s"),
    compiler_params=pltpu.CompilerParams(needs_layout_passes=False),
)


def _segment_sum(values, dst, num_out):
    N = values.shape[0]
    M = num_out
    nchunks = 2 * max(1, -(-M // (2 * 12800)))
    CH = -(-(-(-M // nchunks)) // 128) * 128
    Mpad = nchunks * CH
    SEG = 2048
    nseg = -(-(-(-N // 16)) // SEG)
    ept = nseg * SEG
    Npad = 16 * ept
    idx_p = jnp.pad(dst, (0, Npad - N), constant_values=Mpad)
    rpt = CH // 16

    def body(vals_hbm, idx_hbm, out_hbm,
             idx_v, posb, dlb, rows_v, b0_v, b1_v, zero_v, sem0, sem1, acc_sh):
        core = lax.axis_index("c")
        t = lax.axis_index("s")
        zf = jnp.zeros((16,), jnp.float32)
        for r in range(16):
            for j in range(8):
                zero_v[r, pl.ds(j * 16, 16)] = zf
        lanes = lax.iota(jnp.int32, 16)

        def chunk_body(ci, ccarry):
            lo = ci * CH

            @pl.when(core == (ci % 2))
            def _():
                r0 = t * rpt
                nz, remz = divmod(rpt, 16)
                for k in range(nz):
                    pltpu.sync_copy(zero_v, acc_sh.at[pl.ds(r0 + k * 16, 16)])
                if remz:
                    pltpu.sync_copy(zero_v.at[pl.ds(0, remz)],
                                    acc_sh.at[pl.ds(r0 + nz * 16, remz)])
                plsc.subcore_barrier()

                def seg_body(sg, scarry):
                    pltpu.sync_copy(
                        idx_hbm.at[pl.ds(t * ept + sg * SEG, SEG)], idx_v)
                    pos0 = t * ept + sg * SEG

                    def scan(i, off):
                        v = idx_v[pl.ds(i * 16, 16)]
                        inr = (v >= lo) & (v < lo + CH)
                        plsc.store_compressed(posb.at[pl.ds(off, 16)],
                                              pos0 + i * 16 + lanes, mask=inr)
                        plsc.store_compressed(dlb.at[pl.ds(off, 16)],
                                              v - lo, mask=inr)
                        return off + plsc.all_reduce_population_count(inr)[0]

                    off = lax.fori_loop(0, SEG // 16, scan, 0)
                    posb[pl.ds(off, 16)] = jnp.zeros((16,), jnp.int32)
                    dlb[pl.ds(off, 16)] = CH + lanes
                    nact = (off + 15) // 16

                    def gather(b, buf, sem):
                        pltpu.async_copy(
                            vals_hbm.at[posb[pl.ds(b * 16, 16)]], buf, sem)

                    def drain_scatter(b, buf, sem):
                        pltpu.make_async_copy(
                            vals_hbm.at[pl.ds(0, 16)], buf, sem).wait()
                        pltpu.sync_copy(buf, acc_sh.at[dlb[pl.ds(b * 16, 16)]],
                                        add=True)

                    @pl.when(nact > 0)
                    def _():
                        gather(0, b0_v, sem0)

                    def flush(jj, fcarry):
                        b0 = 2 * jj

                        @pl.when(b0 + 1 < nact)
                        def _():
                            gather(b0 + 1, b1_v, sem1)
                        drain_scatter(b0, b0_v, sem0)

                        @pl.when(b0 + 2 < nact)
                        def _():
                            gather(b0 + 2, b0_v, sem0)

                        @pl.when(b0 + 1 < nact)
                        def _():
                            drain_scatter(b0 + 1, b1_v, sem1)
                        return fcarry

                    lax.fori_loop(0, (nact + 1) // 2, flush, 0)
                    return scarry

                lax.fori_loop(0, nseg, seg_body, 0)
                plsc.subcore_barrier()

                nc2, rem2 = divmod(rpt, 64)
                for k in range(nc2):
                    pltpu.sync_copy(acc_sh.at[pl.ds(r0 + k * 64, 64)], rows_v)
                    pltpu.sync_copy(rows_v,
                                    out_hbm.at[pl.ds(lo + r0 + k * 64, 64)])
                if rem2:
                    pltpu.sync_copy(acc_sh.at[pl.ds(r0 + nc2 * 64, rem2)],
                                    rows_v.at[pl.ds(0, rem2)])
                    pltpu.sync_copy(rows_v.at[pl.ds(0, rem2)],
                                    out_hbm.at[pl.ds(lo + r0 + nc2 * 64, rem2)])
            return ccarry

        lax.fori_loop(0, nchunks, chunk_body, 0)

    k = pl.kernel(
        body,
        out_type=jax.ShapeDtypeStruct((Mpad, D), jnp.float32),
        scratch_types=[
            pltpu.VMEM((SEG,), jnp.int32),
            pltpu.VMEM((SEG + 16,), jnp.int32),
            pltpu.VMEM((SEG + 16,), jnp.int32),
            pltpu.VMEM((64, D), jnp.float32),
            pltpu.VMEM((16, D), jnp.float32),
            pltpu.VMEM((16, D), jnp.float32),
            pltpu.VMEM((16, D), jnp.float32),
            pltpu.SemaphoreType.DMA,
            pltpu.SemaphoreType.DMA,
            pltpu.VMEM_SHARED((CH + 64, D), jnp.float32),
        ],
        **_MESH,
    )
    return k(values, idx_p)


def _gather_add(table, src, dst):
    E = src.shape[0]
    B = 256
    per_w = -(-E // (32 * B)) * B
    Epad = per_w * 32
    nblk = per_w // B
    src_p = jnp.pad(src, (0, Epad - E))
    dst_p = jnp.pad(dst, (0, Epad - E))

    def body(table_hbm, src_hbm, dst_hbm, out_hbm, sidx_v, didx_v, buf_v):
        core = lax.axis_index("c")
        s = lax.axis_index("s")
        wid = s * 2 + core
        base = wid * per_w
        pltpu.sync_copy(src_hbm.at[pl.ds(base, per_w)], sidx_v)
        pltpu.sync_copy(dst_hbm.at[pl.ds(base, per_w)], didx_v)

        def blk(i, carry):
            for h in range(0, B, 128):
                pltpu.sync_copy(table_hbm.at[sidx_v.at[pl.ds(i * B + h, 128)]],
                                buf_v.at[pl.ds(h, 128)])
                pltpu.sync_copy(table_hbm.at[didx_v.at[pl.ds(i * B + h, 128)]],
                                buf_v.at[pl.ds(h, 128)], add=True)
            pltpu.sync_copy(buf_v, out_hbm.at[pl.ds(base + i * B, B)])
            return carry

        lax.fori_loop(0, nblk, blk, 0)

    k = pl.kernel(
        body,
        out_type=jax.ShapeDtypeStruct((Epad, D), jnp.float32),
        scratch_types=[
            pltpu.VMEM((per_w,), jnp.int32),
            pltpu.VMEM((per_w,), jnp.int32),
            pltpu.VMEM((B, D), jnp.float32),
        ],
        **_MESH,
    )
    return k(table, src_p, dst_p)


def kernel(n_feat_0, n_feat_1, n_feat_2, n_feat_3, e_feat_0,
           dst_1, dst_2, dst_3, n2e_src, n2e_dst,
           nW1, nW2, nW3, nb1, nb3, nng, nnb, nog, nob,
           eW1, eW2, eW3, eb1, eb3, eng, enb, bng, bnb):
    e1 = _tc_e1(e_feat_0, eW1, eb1)
    agg = _segment_sum(n_feat_3, dst_3, n_feat_2.shape[0])
    r2 = _tc_mlp(n_feat_2, agg, nW1[2], nW2[2], nb1[2], nng[2], nnb[2],
                 nW3[2], nb3[2], nog[2], nob[2])
    agg = _segment_sum(r2, dst_2, n_feat_1.shape[0])
    r1 = _tc_mlp(n_feat_1, agg, nW1[1], nW2[1], nb1[1], nng[1], nnb[1],
                 nW3[1], nb3[1], nog[1], nob[1])
    agg = _segment_sum(r1, dst_1, n_feat_0.shape[0])
    top_repr, top2 = _tc_mlp_top(n_feat_0, agg, nW1[0], nW2[0], nb1[0],
                                 nng[0], nnb[0], nW3[0], nb3[0], nog[0],
                                 nob[0], eW2)
    n2e2 = _gather_add(top2, n2e_src, n2e_dst)
    edge_repr = _tc_edge(e1, n2e2, eng, enb, eW3, eb3, bng, bnb)
    return (top_repr, edge_repr)


# speedup vs baseline: 1.0434x; 1.0434x over previous
import functools

import jax
import jax.numpy as jnp
from jax import lax
from jax.experimental import pallas as pl
from jax.experimental.pallas import tpu as pltpu
from jax.experimental.pallas import tpu_sc as plsc

D = 128
_R = 1000


def _ln(h, g, b):
    m = jnp.mean(h, axis=-1, keepdims=True)
    v = jnp.mean((h - m) ** 2, axis=-1, keepdims=True)
    return (h - m) * lax.rsqrt(v + 1e-5) * g + b


def _mlp_body(x_ref, a_ref, W1_ref, W2_ref, b1_ref, ng_ref, nb_ref, W3_ref,
              b3_ref, og_ref, ob_ref, o_ref):
    h = (jnp.dot(x_ref[...], W1_ref[...], preferred_element_type=jnp.float32)
         + jnp.dot(a_ref[...], W2_ref[...], preferred_element_type=jnp.float32)
         + b1_ref[...])
    h = _ln(h, ng_ref[...], nb_ref[...])
    h = jnp.maximum(h, 0.0)
    h = jnp.dot(h, W3_ref[...], preferred_element_type=jnp.float32) + b3_ref[...]
    h = _ln(h, og_ref[...], ob_ref[...])
    o_ref[...] = jnp.maximum(h, 0.0)


def _mlp_top_body(x_ref, a_ref, W1_ref, W2_ref, b1_ref, ng_ref, nb_ref, W3_ref,
                  b3_ref, og_ref, ob_ref, eW2_ref, o_ref, o2_ref):
    _mlp_body(x_ref, a_ref, W1_ref, W2_ref, b1_ref, ng_ref, nb_ref, W3_ref,
              b3_ref, og_ref, ob_ref, o_ref)
    o2_ref[...] = jnp.dot(o_ref[...], eW2_ref[...],
                          preferred_element_type=jnp.float32)


def _edge_body(e_ref, g_ref, W1_ref, b1_ref, ng_ref, nb_ref, W3_ref, b3_ref,
               bg_ref, bb_ref, o_ref):
    h = (jnp.dot(e_ref[...], W1_ref[...], preferred_element_type=jnp.float32)
         + g_ref[...] + b1_ref[...])
    h = _ln(h, ng_ref[...], nb_ref[...])
    h = jnp.maximum(h, 0.0)
    h = jnp.dot(h, W3_ref[...], preferred_element_type=jnp.float32) + b3_ref[...]
    h = h * bg_ref[...] + bb_ref[...]
    o_ref[...] = jnp.maximum(h, 0.0)


_row_spec = pl.BlockSpec((_R, D), lambda i: (i, 0))
_w_spec = pl.BlockSpec((D, D), lambda i: (0, 0))
_v_spec = pl.BlockSpec((1, D), lambda i: (0, 0))


def _tc_mlp(x, agg, W1, W2, b1, ng, nb, W3, b3, og, ob):
    M = x.shape[0]
    return pl.pallas_call(
        _mlp_body,
        grid=(M // _R,),
        in_specs=[_row_spec, _row_spec, _w_spec, _w_spec, _v_spec, _v_spec,
                  _v_spec, _w_spec, _v_spec, _v_spec, _v_spec],
        out_specs=_row_spec,
        out_shape=jax.ShapeDtypeStruct((M, D), jnp.float32),
    )(x, agg, W1, W2, b1.reshape(1, D), ng.reshape(1, D), nb.reshape(1, D),
      W3, b3.reshape(1, D), og.reshape(1, D), ob.reshape(1, D))


def _tc_mlp_top(x, agg, W1, W2, b1, ng, nb, W3, b3, og, ob, eW2):
    M = x.shape[0]
    return pl.pallas_call(
        _mlp_top_body,
        grid=(M // _R,),
        in_specs=[_row_spec, _row_spec, _w_spec, _w_spec, _v_spec, _v_spec,
                  _v_spec, _w_spec, _v_spec, _v_spec, _v_spec, _w_spec],
        out_specs=[_row_spec, _row_spec],
        out_shape=[jax.ShapeDtypeStruct((M, D), jnp.float32),
                   jax.ShapeDtypeStruct((M, D), jnp.float32)],
    )(x, agg, W1, W2, b1.reshape(1, D), ng.reshape(1, D), nb.reshape(1, D),
      W3, b3.reshape(1, D), og.reshape(1, D), ob.reshape(1, D), eW2)


def _tc_edge(e_feat, n2e2, W1, b1, ng, nb, W3, b3, bg, bb):
    M = e_feat.shape[0]
    return pl.pallas_call(
        _edge_body,
        grid=(M // _R,),
        in_specs=[_row_spec, _row_spec, _w_spec, _v_spec, _v_spec, _v_spec,
                  _w_spec, _v_spec, _v_spec, _v_spec],
        out_specs=_row_spec,
        out_shape=jax.ShapeDtypeStruct((M, D), jnp.float32),
    )(e_feat, n2e2, W1, b1.reshape(1, D), ng.reshape(1, D), nb.reshape(1, D),
      W3, b3.reshape(1, D), bg.reshape(1, D), bb.reshape(1, D))


_MESH = dict(
    mesh=plsc.VectorSubcoreMesh(core_axis_name="c", subcore_axis_name="s"),
    compiler_params=pltpu.CompilerParams(needs_layout_passes=False),
)


def _segment_sum(values, dst, num_out):
    N = values.shape[0]
    M = num_out
    nchunks = 2 * max(1, -(-M // (2 * 12800)))
    CH = -(-(-(-M // nchunks)) // 128) * 128
    Mpad = nchunks * CH
    SEG = 2048
    nseg = -(-(-(-N // 16)) // SEG)
    ept = nseg * SEG
    Npad = 16 * ept
    idx_p = jnp.pad(dst, (0, Npad - N), constant_values=Mpad)
    rpt = CH // 16

    def body(vals_hbm, idx_hbm, out_hbm,
             idx_v, posb, dlb, rows_v, b0_v, b1_v, zero_v, sem0, sem1, acc_sh):
        core = lax.axis_index("c")
        t = lax.axis_index("s")
        zf = jnp.zeros((16,), jnp.float32)
        for r in range(16):
            for j in range(8):
                zero_v[r, pl.ds(j * 16, 16)] = zf
        lanes = lax.iota(jnp.int32, 16)

        def chunk_body(ci, ccarry):
            lo = ci * CH

            @pl.when(core == (ci % 2))
            def _():
                r0 = t * rpt
                nz, remz = divmod(rpt, 16)
                for k in range(nz):
                    pltpu.sync_copy(zero_v, acc_sh.at[pl.ds(r0 + k * 16, 16)])
                if remz:
                    pltpu.sync_copy(zero_v.at[pl.ds(0, remz)],
                                    acc_sh.at[pl.ds(r0 + nz * 16, remz)])
                plsc.subcore_barrier()

                def seg_body(sg, scarry):
                    pltpu.sync_copy(
                        idx_hbm.at[pl.ds(t * ept + sg * SEG, SEG)], idx_v)
                    pos0 = t * ept + sg * SEG

                    def scan(i, off):
                        v = idx_v[pl.ds(i * 16, 16)]
                        inr = (v >= lo) & (v < lo + CH)
                        plsc.store_compressed(posb.at[pl.ds(off, 16)],
                                              pos0 + i * 16 + lanes, mask=inr)
                        plsc.store_compressed(dlb.at[pl.ds(off, 16)],
                                              v - lo, mask=inr)
                        return off + plsc.all_reduce_population_count(inr)[0]

                    off = lax.fori_loop(0, SEG // 16, scan, 0)
                    posb[pl.ds(off, 16)] = jnp.zeros((16,), jnp.int32)
                    dlb[pl.ds(off, 16)] = CH + lanes
                    nact = (off + 15) // 16

                    def gather(b, buf, sem):
                        pltpu.async_copy(
                            vals_hbm.at[posb[pl.ds(b * 16, 16)]], buf, sem)

                    def drain_scatter(b, buf, sem):
                        pltpu.make_async_copy(
                            vals_hbm.at[pl.ds(0, 16)], buf, sem).wait()
                        pltpu.sync_copy(buf, acc_sh.at[dlb[pl.ds(b * 16, 16)]],
                                        add=True)

                    @pl.when(nact > 0)
                    def _():
                        gather(0, b0_v, sem0)

                    def flush(jj, fcarry):
                        b0 = 2 * jj

                        @pl.when(b0 + 1 < nact)
                        def _():
                            gather(b0 + 1, b1_v, sem1)
                        drain_scatter(b0, b0_v, sem0)

                        @pl.when(b0 + 2 < nact)
                        def _():
                            gather(b0 + 2, b0_v, sem0)

                        @pl.when(b0 + 1 < nact)
                        def _():
                            drain_scatter(b0 + 1, b1_v, sem1)
                        return fcarry

                    lax.fori_loop(0, (nact + 1) // 2, flush, 0)
                    return scarry

                lax.fori_loop(0, nseg, seg_body, 0)
                plsc.subcore_barrier()

                nc2, rem2 = divmod(rpt, 64)
                for k in range(nc2):
                    pltpu.sync_copy(acc_sh.at[pl.ds(r0 + k * 64, 64)], rows_v)
                    pltpu.sync_copy(rows_v,
                                    out_hbm.at[pl.ds(lo + r0 + k * 64, 64)])
                if rem2:
                    pltpu.sync_copy(acc_sh.at[pl.ds(r0 + nc2 * 64, rem2)],
                                    rows_v.at[pl.ds(0, rem2)])
                    pltpu.sync_copy(rows_v.at[pl.ds(0, rem2)],
                                    out_hbm.at[pl.ds(lo + r0 + nc2 * 64, rem2)])
            return ccarry

        lax.fori_loop(0, nchunks, chunk_body, 0)

    k = pl.kernel(
        body,
        out_type=jax.ShapeDtypeStruct((Mpad, D), jnp.float32),
        scratch_types=[
            pltpu.VMEM((SEG,), jnp.int32),
            pltpu.VMEM((SEG + 16,), jnp.int32),
            pltpu.VMEM((SEG + 16,), jnp.int32),
            pltpu.VMEM((64, D), jnp.float32),
            pltpu.VMEM((16, D), jnp.float32),
            pltpu.VMEM((16, D), jnp.float32),
            pltpu.VMEM((16, D), jnp.float32),
            pltpu.SemaphoreType.DMA,
            pltpu.SemaphoreType.DMA,
            pltpu.VMEM_SHARED((CH + 64, D), jnp.float32),
        ],
        **_MESH,
    )
    return k(values, idx_p)


def _gather_add(table, src, dst):
    E = src.shape[0]
    B = 256
    per_w = -(-E // (32 * 2 * B)) * (2 * B)
    Epad = per_w * 32
    nblk = per_w // B
    src_p = jnp.pad(src, (0, Epad - E))
    dst_p = jnp.pad(dst, (0, Epad - E))

    def body(table_hbm, src_hbm, dst_hbm, out_hbm, sidx_v, didx_v,
             buf0_v, buf1_v, sem0, sem1):
        core = lax.axis_index("c")
        s = lax.axis_index("s")
        wid = s * 2 + core
        base = wid * per_w
        pltpu.sync_copy(src_hbm.at[pl.ds(base, per_w)], sidx_v)
        pltpu.sync_copy(dst_hbm.at[pl.ds(base, per_w)], didx_v)

        def fill(i, buf):
            for h in range(0, B, 128):
                pltpu.sync_copy(table_hbm.at[sidx_v.at[pl.ds(i * B + h, 128)]],
                                buf.at[pl.ds(h, 128)])
                pltpu.sync_copy(table_hbm.at[didx_v.at[pl.ds(i * B + h, 128)]],
                                buf.at[pl.ds(h, 128)], add=True)

        def wait_store(buf, sem):
            pltpu.make_async_copy(buf, out_hbm.at[pl.ds(base, B)], sem).wait()

        def blk2(j, carry):
            i0 = 2 * j

            @pl.when(j > 0)
            def _():
                wait_store(buf0_v, sem0)
            fill(i0, buf0_v)
            pltpu.async_copy(buf0_v, out_hbm.at[pl.ds(base + i0 * B, B)], sem0)

            @pl.when(j > 0)
            def _():
                wait_store(buf1_v, sem1)
            fill(i0 + 1, buf1_v)
            pltpu.async_copy(buf1_v, out_hbm.at[pl.ds(base + (i0 + 1) * B, B)],
                             sem1)
            return carry

        lax.fori_loop(0, nblk // 2, blk2, 0)
        wait_store(buf0_v, sem0)
        wait_store(buf1_v, sem1)

    k = pl.kernel(
        body,
        out_type=jax.ShapeDtypeStruct((Epad, D), jnp.float32),
        scratch_types=[
            pltpu.VMEM((per_w,), jnp.int32),
            pltpu.VMEM((per_w,), jnp.int32),
            pltpu.VMEM((B, D), jnp.float32),
            pltpu.VMEM((B, D), jnp.float32),
            pltpu.SemaphoreType.DMA,
            pltpu.SemaphoreType.DMA,
        ],
        **_MESH,
    )
    return k(table, src_p, dst_p)


def kernel(n_feat_0, n_feat_1, n_feat_2, n_feat_3, e_feat_0,
           dst_1, dst_2, dst_3, n2e_src, n2e_dst,
           nW1, nW2, nW3, nb1, nb3, nng, nnb, nog, nob,
           eW1, eW2, eW3, eb1, eb3, eng, enb, bng, bnb):
    agg = _segment_sum(n_feat_3, dst_3, n_feat_2.shape[0])
    r2 = _tc_mlp(n_feat_2, agg, nW1[2], nW2[2], nb1[2], nng[2], nnb[2],
                 nW3[2], nb3[2], nog[2], nob[2])
    agg = _segment_sum(r2, dst_2, n_feat_1.shape[0])
    r1 = _tc_mlp(n_feat_1, agg, nW1[1], nW2[1], nb1[1], nng[1], nnb[1],
                 nW3[1], nb3[1], nog[1], nob[1])
    agg = _segment_sum(r1, dst_1, n_feat_0.shape[0])
    top_repr, top2 = _tc_mlp_top(n_feat_0, agg, nW1[0], nW2[0], nb1[0],
                                 nng[0], nnb[0], nW3[0], nb3[0], nog[0],
                                 nob[0], eW2)
    n2e2 = _gather_add(top2, n2e_src, n2e_dst)
    edge_repr = _tc_edge(e_feat_0, n2e2, eW1, eb1, eng, enb, eW3, eb3,
                         bng, bnb)
    return (top_repr, edge_repr)


# speedup vs baseline: 1.0525x; 1.0087x over previous
import functools

import jax
import jax.numpy as jnp
from jax import lax
from jax.experimental import pallas as pl
from jax.experimental.pallas import tpu as pltpu
from jax.experimental.pallas import tpu_sc as plsc

D = 128
_R = 1000


def _ln(h, g, b):
    m = jnp.mean(h, axis=-1, keepdims=True)
    v = jnp.mean((h - m) ** 2, axis=-1, keepdims=True)
    return (h - m) * lax.rsqrt(v + 1e-5) * g + b


def _mlp_body(x_ref, a_ref, W1_ref, W2_ref, b1_ref, ng_ref, nb_ref, W3_ref,
              b3_ref, og_ref, ob_ref, o_ref):
    h = (jnp.dot(x_ref[...], W1_ref[...], preferred_element_type=jnp.float32)
         + jnp.dot(a_ref[...], W2_ref[...], preferred_element_type=jnp.float32)
         + b1_ref[...])
    h = _ln(h, ng_ref[...], nb_ref[...])
    h = jnp.maximum(h, 0.0)
    h = jnp.dot(h, W3_ref[...], preferred_element_type=jnp.float32) + b3_ref[...]
    h = _ln(h, og_ref[...], ob_ref[...])
    o_ref[...] = jnp.maximum(h, 0.0)


def _mlp_top_body(x_ref, a_ref, W1_ref, W2_ref, b1_ref, ng_ref, nb_ref, W3_ref,
                  b3_ref, og_ref, ob_ref, eW2_ref, o_ref, o2_ref):
    _mlp_body(x_ref, a_ref, W1_ref, W2_ref, b1_ref, ng_ref, nb_ref, W3_ref,
              b3_ref, og_ref, ob_ref, o_ref)
    o2_ref[...] = jnp.dot(o_ref[...], eW2_ref[...],
                          preferred_element_type=jnp.float32)


def _edge_body(e_ref, g_ref, W1_ref, b1_ref, ng_ref, nb_ref, W3_ref, b3_ref,
               bg_ref, bb_ref, o_ref):
    h = (jnp.dot(e_ref[...], W1_ref[...], preferred_element_type=jnp.float32)
         + g_ref[...] + b1_ref[...])
    h = _ln(h, ng_ref[...], nb_ref[...])
    h = jnp.maximum(h, 0.0)
    h = jnp.dot(h, W3_ref[...], preferred_element_type=jnp.float32) + b3_ref[...]
    h = h * bg_ref[...] + bb_ref[...]
    o_ref[...] = jnp.maximum(h, 0.0)


_row_spec = pl.BlockSpec((_R, D), lambda i: (i, 0))
_w_spec = pl.BlockSpec((D, D), lambda i: (0, 0))
_v_spec = pl.BlockSpec((1, D), lambda i: (0, 0))


def _tc_mlp(x, agg, W1, W2, b1, ng, nb, W3, b3, og, ob):
    M = x.shape[0]
    return pl.pallas_call(
        _mlp_body,
        grid=(M // _R,),
        in_specs=[_row_spec, _row_spec, _w_spec, _w_spec, _v_spec, _v_spec,
                  _v_spec, _w_spec, _v_spec, _v_spec, _v_spec],
        out_specs=_row_spec,
        out_shape=jax.ShapeDtypeStruct((M, D), jnp.float32),
    )(x, agg, W1, W2, b1.reshape(1, D), ng.reshape(1, D), nb.reshape(1, D),
      W3, b3.reshape(1, D), og.reshape(1, D), ob.reshape(1, D))


def _tc_mlp_top(x, agg, W1, W2, b1, ng, nb, W3, b3, og, ob, eW2):
    M = x.shape[0]
    return pl.pallas_call(
        _mlp_top_body,
        grid=(M // _R,),
        in_specs=[_row_spec, _row_spec, _w_spec, _w_spec, _v_spec, _v_spec,
                  _v_spec, _w_spec, _v_spec, _v_spec, _v_spec, _w_spec],
        out_specs=[_row_spec, _row_spec],
        out_shape=[jax.ShapeDtypeStruct((M, D), jnp.float32),
                   jax.ShapeDtypeStruct((M, D), jnp.float32)],
    )(x, agg, W1, W2, b1.reshape(1, D), ng.reshape(1, D), nb.reshape(1, D),
      W3, b3.reshape(1, D), og.reshape(1, D), ob.reshape(1, D), eW2)


def _tc_edge(e_feat, n2e2, W1, b1, ng, nb, W3, b3, bg, bb):
    M = e_feat.shape[0]
    return pl.pallas_call(
        _edge_body,
        grid=(M // _R,),
        in_specs=[_row_spec, _row_spec, _w_spec, _v_spec, _v_spec, _v_spec,
                  _w_spec, _v_spec, _v_spec, _v_spec],
        out_specs=_row_spec,
        out_shape=jax.ShapeDtypeStruct((M, D), jnp.float32),
    )(e_feat, n2e2, W1, b1.reshape(1, D), ng.reshape(1, D), nb.reshape(1, D),
      W3, b3.reshape(1, D), bg.reshape(1, D), bb.reshape(1, D))


_MESH = dict(
    mesh=plsc.VectorSubcoreMesh(core_axis_name="c", subcore_axis_name="s"),
    compiler_params=pltpu.CompilerParams(needs_layout_passes=False),
)


def _segment_sum(values, dst, num_out):
    N = values.shape[0]
    M = num_out
    nchunks = 2 * max(1, -(-M // (2 * 12800)))
    CH = -(-(-(-M // nchunks)) // 128) * 128
    Mpad = nchunks * CH
    SEG = 2048
    nseg = -(-(-(-N // 16)) // SEG)
    ept = nseg * SEG
    Npad = 16 * ept
    idx_p = jnp.pad(dst, (0, Npad - N), constant_values=Mpad)
    rpt = CH // 16

    def body(vals_hbm, idx_hbm, out_hbm,
             idx_v, posb, dlb, rows_v, b0_v, b1_v, zero_v, sem0, sem1, acc_sh):
        core = lax.axis_index("c")
        t = lax.axis_index("s")
        zf = jnp.zeros((16,), jnp.float32)
        for r in range(16):
            for j in range(8):
                zero_v[r, pl.ds(j * 16, 16)] = zf
        lanes = lax.iota(jnp.int32, 16)
        pltpu.sync_copy(idx_hbm.at[pl.ds(t * ept, ept)], idx_v)

        def chunk_body(ci, ccarry):
            lo = ci * CH

            @pl.when(core == (ci % 2))
            def _():
                r0 = t * rpt
                nz, remz = divmod(rpt, 16)
                for k in range(nz):
                    pltpu.sync_copy(zero_v, acc_sh.at[pl.ds(r0 + k * 16, 16)])
                if remz:
                    pltpu.sync_copy(zero_v.at[pl.ds(0, remz)],
                                    acc_sh.at[pl.ds(r0 + nz * 16, remz)])
                plsc.subcore_barrier()

                def seg_body(sg, scarry):
                    e0 = sg * SEG

                    def scan(i, off):
                        v = idx_v[pl.ds(e0 + i * 16, 16)]
                        inr = (v >= lo) & (v < lo + CH)
                        plsc.store_compressed(posb.at[pl.ds(off, 16)],
                                              t * ept + e0 + i * 16 + lanes,
                                              mask=inr)
                        plsc.store_compressed(dlb.at[pl.ds(off, 16)],
                                              v - lo, mask=inr)
                        return off + plsc.all_reduce_population_count(inr)[0]

                    off = lax.fori_loop(0, SEG // 16, scan, 0)
                    posb[pl.ds(off, 16)] = jnp.zeros((16,), jnp.int32)
                    dlb[pl.ds(off, 16)] = CH + lanes
                    nact = (off + 15) // 16

                    def gather(b, buf, sem):
                        pltpu.async_copy(
                            vals_hbm.at[posb[pl.ds(b * 16, 16)]], buf, sem)

                    def drain_scatter(b, buf, sem):
                        pltpu.make_async_copy(
                            vals_hbm.at[pl.ds(0, 16)], buf, sem).wait()
                        pltpu.sync_copy(buf, acc_sh.at[dlb[pl.ds(b * 16, 16)]],
                                        add=True)

                    @pl.when(nact > 0)
                    def _():
                        gather(0, b0_v, sem0)

                    def flush(jj, fcarry):
                        b0 = 2 * jj

                        @pl.when(b0 + 1 < nact)
                        def _():
                            gather(b0 + 1, b1_v, sem1)
                        drain_scatter(b0, b0_v, sem0)

                        @pl.when(b0 + 2 < nact)
                        def _():
                            gather(b0 + 2, b0_v, sem0)

                        @pl.when(b0 + 1 < nact)
                        def _():
                            drain_scatter(b0 + 1, b1_v, sem1)
                        return fcarry

                    lax.fori_loop(0, (nact + 1) // 2, flush, 0)
                    return scarry

                lax.fori_loop(0, nseg, seg_body, 0)
                plsc.subcore_barrier()

                nc2, rem2 = divmod(rpt, 64)
                for k in range(nc2):
                    pltpu.sync_copy(acc_sh.at[pl.ds(r0 + k * 64, 64)], rows_v)
                    pltpu.sync_copy(rows_v,
                                    out_hbm.at[pl.ds(lo + r0 + k * 64, 64)])
                if rem2:
                    pltpu.sync_copy(acc_sh.at[pl.ds(r0 + nc2 * 64, rem2)],
                                    rows_v.at[pl.ds(0, rem2)])
                    pltpu.sync_copy(rows_v.at[pl.ds(0, rem2)],
                                    out_hbm.at[pl.ds(lo + r0 + nc2 * 64, rem2)])
            return ccarry

        lax.fori_loop(0, nchunks, chunk_body, 0)

    k = pl.kernel(
        body,
        out_type=jax.ShapeDtypeStruct((Mpad, D), jnp.float32),
        scratch_types=[
            pltpu.VMEM((ept,), jnp.int32),
            pltpu.VMEM((SEG + 16,), jnp.int32),
            pltpu.VMEM((SEG + 16,), jnp.int32),
            pltpu.VMEM((64, D), jnp.float32),
            pltpu.VMEM((16, D), jnp.float32),
            pltpu.VMEM((16, D), jnp.float32),
            pltpu.VMEM((16, D), jnp.float32),
            pltpu.SemaphoreType.DMA,
            pltpu.SemaphoreType.DMA,
            pltpu.VMEM_SHARED((CH + 64, D), jnp.float32),
        ],
        **_MESH,
    )
    return k(values, idx_p)


def _gather_add(table, src, dst):
    E = src.shape[0]
    B = 256
    per_w = -(-E // (32 * 2 * B)) * (2 * B)
    Epad = per_w * 32
    nblk = per_w // B
    src_p = jnp.pad(src, (0, Epad - E))
    dst_p = jnp.pad(dst, (0, Epad - E))

    def body(table_hbm, src_hbm, dst_hbm, out_hbm, sidx_v, didx_v,
             buf0_v, buf1_v, sem0, sem1):
        core = lax.axis_index("c")
        s = lax.axis_index("s")
        wid = s * 2 + core
        base = wid * per_w
        pltpu.sync_copy(src_hbm.at[pl.ds(base, per_w)], sidx_v)
        pltpu.sync_copy(dst_hbm.at[pl.ds(base, per_w)], didx_v)

        def fill(i, buf):
            for h in range(0, B, 128):
                pltpu.sync_copy(table_hbm.at[sidx_v.at[pl.ds(i * B + h, 128)]],
                                buf.at[pl.ds(h, 128)])
                pltpu.sync_copy(table_hbm.at[didx_v.at[pl.ds(i * B + h, 128)]],
                                buf.at[pl.ds(h, 128)], add=True)

        def wait_store(buf, sem):
            pltpu.make_async_copy(buf, out_hbm.at[pl.ds(base, B)], sem).wait()

        def blk2(j, carry):
            i0 = 2 * j

            @pl.when(j > 0)
            def _():
                wait_store(buf0_v, sem0)
            fill(i0, buf0_v)
            pltpu.async_copy(buf0_v, out_hbm.at[pl.ds(base + i0 * B, B)], sem0)

            @pl.when(j > 0)
            def _():
                wait_store(buf1_v, sem1)
            fill(i0 + 1, buf1_v)
            pltpu.async_copy(buf1_v, out_hbm.at[pl.ds(base + (i0 + 1) * B, B)],
                             sem1)
            return carry

        lax.fori_loop(0, nblk // 2, blk2, 0)
        wait_store(buf0_v, sem0)
        wait_store(buf1_v, sem1)

    k = pl.kernel(
        body,
        out_type=jax.ShapeDtypeStruct((Epad, D), jnp.float32),
        scratch_types=[
            pltpu.VMEM((per_w,), jnp.int32),
            pltpu.VMEM((per_w,), jnp.int32),
            pltpu.VMEM((B, D), jnp.float32),
            pltpu.VMEM((B, D), jnp.float32),
            pltpu.SemaphoreType.DMA,
            pltpu.SemaphoreType.DMA,
        ],
        **_MESH,
    )
    return k(table, src_p, dst_p)


def kernel(n_feat_0, n_feat_1, n_feat_2, n_feat_3, e_feat_0,
           dst_1, dst_2, dst_3, n2e_src, n2e_dst,
           nW1, nW2, nW3, nb1, nb3, nng, nnb, nog, nob,
           eW1, eW2, eW3, eb1, eb3, eng, enb, bng, bnb):
    agg = _segment_sum(n_feat_3, dst_3, n_feat_2.shape[0])
    r2 = _tc_mlp(n_feat_2, agg, nW1[2], nW2[2], nb1[2], nng[2], nnb[2],
                 nW3[2], nb3[2], nog[2], nob[2])
    agg = _segment_sum(r2, dst_2, n_feat_1.shape[0])
    r1 = _tc_mlp(n_feat_1, agg, nW1[1], nW2[1], nb1[1], nng[1], nnb[1],
                 nW3[1], nb3[1], nog[1], nob[1])
    agg = _segment_sum(r1, dst_1, n_feat_0.shape[0])
    top_repr, top2 = _tc_mlp_top(n_feat_0, agg, nW1[0], nW2[0], nb1[0],
                                 nng[0], nnb[0], nW3[0], nb3[0], nog[0],
                                 nob[0], eW2)
    n2e2 = _gather_add(top2, n2e_src, n2e_dst)
    edge_repr = _tc_edge(e_feat_0, n2e2, eW1, eb1, eng, enb, eW3, eb3,
                         bng, bnb)
    return (top_repr, edge_repr)
